# baseline (speedup 1.0000x reference)
import functools

import jax
import jax.numpy as jnp
import numpy as np
from jax import lax
from jax.experimental import pallas as pl
from jax.experimental.pallas import tpu as pltpu
from jax.experimental.pallas import tpu_sc as plsc

LB = 128


@functools.cache
def _build_gather(b, sc, vocab, d, nw, nc):
    nblk = (b // LB) * sc
    bpw = nblk // nw
    kb_per_s = b // LB

    mesh = plsc.VectorSubcoreMesh(core_axis_name="c", subcore_axis_name="s")

    assert bpw % 2 == 0 and bpw >= 6

    @functools.partial(
        pl.kernel,
        mesh=mesh,
        compiler_params=pltpu.CompilerParams(use_tc_tiling_on_sc=False),
        out_type=jax.ShapeDtypeStruct((sc, b, d), jnp.float32),
        scratch_types=[
            pltpu.VMEM((bpw, LB), jnp.int32),
            pltpu.VMEM((2, LB, d), jnp.float32),
            pltpu.SemaphoreType.DMA,
            pltpu.SemaphoreType.DMA,
            pltpu.SemaphoreType.DMA,
            pltpu.SemaphoreType.DMA,
        ],
    )
    def emb(table_h, idx_h, out_h, idx_v, rows_v, gsem0, gsem1, osem0, osem1):
        ci = lax.axis_index("c")
        si = lax.axis_index("s")
        wid = si * nc + ci
        gsems = (gsem0, gsem1)
        osems = (osem0, osem1)
        pltpu.sync_copy(idx_h.at[pl.ds(wid * bpw, bpw)], idx_v)

        def gather(t, bf):
            pltpu.async_copy(table_h.at[idx_v.at[t]], rows_v.at[bf], gsems[bf])

        def gather_wait(bf):
            pltpu.make_async_copy(
                table_h.at[idx_v.at[0]], rows_v.at[bf], gsems[bf]
            ).wait()

        def out_start(t, bf):
            blk = wid * bpw + t
            s_i = blk // kb_per_s
            kb = blk % kb_per_s
            pltpu.async_copy(
                rows_v.at[bf], out_h.at[s_i, pl.ds(kb * LB, LB)], osems[bf]
            )

        def out_wait(bf):
            pltpu.make_async_copy(
                rows_v.at[bf], out_h.at[0, pl.ds(0, LB)], osems[bf]
            ).wait()

        gather(0, 0)
        gather(1, 1)

        def body(k, carry):
            t0 = 2 * k
            for bf in range(2):
                gather_wait(bf)
                out_start(t0 + bf, bf)
                out_wait(bf)
                gather(t0 + 2 + bf, bf)
            return carry

        lax.fori_loop(0, bpw // 2 - 1, body, 0)

        for bf in range(2):
            gather_wait(bf)
            out_start(bpw - 2 + bf, bf)
        for bf in range(2):
            out_wait(bf)

    return emb


def _tc_block(in_ref, out_ref):
    xt = in_ref[0].T
    kc = in_ref.shape[1] // 64
    for k in range(kc):
        lo, hi = k * 64, (k + 1) * 64
        out_ref[0, :, k, :, 0:64] = xt[0:64, lo:hi].reshape(8, 8, 64)
        out_ref[0, :, k, :, 64:128] = xt[64:128, lo:hi].reshape(8, 8, 64)


def _tc_block2(in_ref, prev_ref, out_ref):
    del prev_ref
    _tc_block(in_ref, out_ref)


@functools.cache
def _build_format(s, sc, s_off, b, d, aliased):
    kb_per_s = b // LB
    td = d // 8
    rows = b * d // LB
    blk_rows = 2048
    grid = (sc, rows // blk_rows)
    kc = blk_rows // 64

    x_spec = pl.BlockSpec((1, blk_rows, LB), lambda i, j: (i, j, 0))
    out_spec = pl.BlockSpec(
        (1, td, kc, 8, LB), lambda i, j: (i + s_off, 0, j, 0, 0)
    )
    out_shape = jax.ShapeDtypeStruct((s, td, kb_per_s, 8, LB), jnp.float32)
    if not aliased:
        return pl.pallas_call(
            _tc_block,
            grid=grid,
            in_specs=[x_spec],
            out_specs=out_spec,
            out_shape=out_shape,
        )
    return pl.pallas_call(
        _tc_block2,
        grid=grid,
        in_specs=[x_spec, pl.BlockSpec(memory_space=pltpu.MemorySpace.HBM)],
        out_specs=out_spec,
        out_shape=out_shape,
        input_output_aliases={1: 0},
    )


def _lane_perm():
    j = np.arange(LB)
    return jnp.asarray(
        np.where(j % 2 == 0, j // 2, LB // 2 + j // 2), jnp.int32
    )


def kernel(sent, table):
    b, s = sent.shape
    vocab, d = table.shape
    nw = 32
    nchunk = 5
    sc = s // nchunk
    idxp = jnp.take(
        sent.astype(jnp.int32).T.reshape((b // LB) * s, LB),
        _lane_perm(),
        axis=1,
    )
    nb2 = (b // LB) * sc
    gather = _build_gather(b, sc, vocab, d, nw, 2)
    gs = [gather(table, idxp[i * nb2 : (i + 1) * nb2]) for i in range(nchunk)]
    fmt = lambda g: g.reshape(sc, b * d // LB, LB)
    r = _build_format(s, sc, 0, b, d, False)(fmt(gs[0]))
    for i in range(1, nchunk):
        r = _build_format(s, sc, i * sc, b, d, True)(fmt(gs[i]), r)
    return r.transpose(2, 4, 0, 1, 3).reshape(b, s, d)


# speedup vs baseline: 1.4436x; 1.4436x over previous
import functools

import jax
import jax.numpy as jnp
import numpy as np
from jax import lax
from jax.experimental import pallas as pl
from jax.experimental.pallas import tpu as pltpu
from jax.experimental.pallas import tpu_sc as plsc

LB = 128


@functools.cache
def _build_gather(b, sc, vocab, d, nw, nc):
    nblk = (b // LB) * sc
    bpw = nblk // nw
    kb_per_s = b // LB

    mesh = plsc.VectorSubcoreMesh(core_axis_name="c", subcore_axis_name="s")

    assert bpw % 2 == 0 and bpw >= 6

    @functools.partial(
        pl.kernel,
        mesh=mesh,
        compiler_params=pltpu.CompilerParams(use_tc_tiling_on_sc=False),
        out_type=jax.ShapeDtypeStruct((sc, b, d), jnp.float32),
        scratch_types=[
            pltpu.VMEM((bpw, LB), jnp.int32),
            pltpu.VMEM((2, LB, d), jnp.float32),
            pltpu.SemaphoreType.DMA,
            pltpu.SemaphoreType.DMA,
            pltpu.SemaphoreType.DMA,
            pltpu.SemaphoreType.DMA,
        ],
    )
    def emb(table_h, idx_h, out_h, idx_v, rows_v, gsem0, gsem1, osem0, osem1):
        ci = lax.axis_index("c")
        si = lax.axis_index("s")
        wid = si * nc + ci
        gsems = (gsem0, gsem1)
        osems = (osem0, osem1)
        pltpu.sync_copy(idx_h.at[pl.ds(wid * bpw, bpw)], idx_v)

        def gather(t, bf):
            pltpu.async_copy(table_h.at[idx_v.at[t]], rows_v.at[bf], gsems[bf])

        def gather_wait(bf):
            pltpu.make_async_copy(
                table_h.at[idx_v.at[0]], rows_v.at[bf], gsems[bf]
            ).wait()

        def out_start(t, bf):
            blk = wid * bpw + t
            s_i = blk // kb_per_s
            kb = blk % kb_per_s
            pltpu.async_copy(
                rows_v.at[bf], out_h.at[s_i, pl.ds(kb * LB, LB)], osems[bf]
            )

        def out_wait(bf):
            pltpu.make_async_copy(
                rows_v.at[bf], out_h.at[0, pl.ds(0, LB)], osems[bf]
            ).wait()

        gather(0, 0)
        gather(1, 1)

        def body(k, carry):
            t0 = 2 * k
            for bf in range(2):
                gather_wait(bf)
                out_start(t0 + bf, bf)
                out_wait(bf)
                gather(t0 + 2 + bf, bf)
            return carry

        lax.fori_loop(0, bpw // 2 - 1, body, 0)

        for bf in range(2):
            gather_wait(bf)
            out_start(bpw - 2 + bf, bf)
        for bf in range(2):
            out_wait(bf)

    return emb


def _tc_block(in_ref, out_ref):
    xt = in_ref[0].T
    kc = in_ref.shape[1] // 64
    for k in range(kc):
        lo, hi = k * 64, (k + 1) * 64
        out_ref[0, :, k, :, 0:64] = xt[0:64, lo:hi].reshape(8, 8, 64)
        out_ref[0, :, k, :, 64:128] = xt[64:128, lo:hi].reshape(8, 8, 64)


def _tc_block2(in_ref, prev_ref, out_ref):
    del prev_ref
    _tc_block(in_ref, out_ref)


@functools.cache
def _build_format(s, sc, s_off, b, d, aliased):
    kb_per_s = b // LB
    td = d // 8
    rows = b * d // LB
    blk_rows = 8192
    grid = (sc, rows // blk_rows)
    kc = blk_rows // 64

    x_spec = pl.BlockSpec((1, blk_rows, LB), lambda i, j: (i, j, 0))
    out_spec = pl.BlockSpec(
        (1, td, kc, 8, LB), lambda i, j: (i + s_off, 0, j, 0, 0)
    )
    out_shape = jax.ShapeDtypeStruct((s, td, kb_per_s, 8, LB), jnp.float32)
    if not aliased:
        return pl.pallas_call(
            _tc_block,
            grid=grid,
            in_specs=[x_spec],
            out_specs=out_spec,
            out_shape=out_shape,
        )
    return pl.pallas_call(
        _tc_block2,
        grid=grid,
        in_specs=[x_spec, pl.BlockSpec(memory_space=pltpu.MemorySpace.HBM)],
        out_specs=out_spec,
        out_shape=out_shape,
        input_output_aliases={1: 0},
    )


def _tpack_block(in_a, in_b, out_ref):
    out_ref[:, 0:64] = in_a[...].T
    out_ref[:, 64:128] = in_b[...].T


TPACK_H = 512000


@functools.cache
def _build_tpack(vocab, d):
    blk = 2048
    grid = (TPACK_H // blk,)
    off = TPACK_H // blk
    last = pl.cdiv(vocab, blk) - 1

    return pl.pallas_call(
        _tpack_block,
        grid=grid,
        in_specs=[
            pl.BlockSpec((d, blk), lambda j: (0, j)),
            pl.BlockSpec((d, blk), lambda j: (0, jnp.minimum(j + off, last))),
        ],
        out_specs=pl.BlockSpec((blk, 2 * d), lambda j: (j, 0)),
        out_shape=jax.ShapeDtypeStruct((TPACK_H, 2 * d), jnp.float32),
    )


def _lane_perm():
    j = np.arange(LB)
    return jnp.asarray(
        np.where(j % 2 == 0, j // 2, LB // 2 + j // 2), jnp.int32
    )


def kernel(sent, table):
    b, s = sent.shape
    vocab, d = table.shape
    nw = 32
    nchunk = 5
    sc = s // nchunk
    r_idx = jnp.take(
        sent.astype(jnp.int32).T.reshape((b // LB) * s, LB),
        _lane_perm(),
        axis=1,
    )
    idxp = jnp.where(
        r_idx < TPACK_H, 2 * r_idx, 2 * (r_idx - TPACK_H) + 1
    )
    nb2 = (b // LB) * sc
    gather = _build_gather(b, sc, 2 * TPACK_H, d, nw, 2)
    tt = table.T
    table_lin = _build_tpack(vocab, d)(tt, tt).reshape(2 * TPACK_H, d)
    gs = [gather(table_lin, idxp[i * nb2 : (i + 1) * nb2]) for i in range(nchunk)]
    fmt = lambda g: g.reshape(sc, b * d // LB, LB)
    r = _build_format(s, sc, 0, b, d, False)(fmt(gs[0]))
    for i in range(1, nchunk):
        r = _build_format(s, sc, i * sc, b, d, True)(fmt(gs[i]), r)
    return r.transpose(2, 4, 0, 1, 3).reshape(b, s, d)
